# baseline (speedup 1.0000x reference)
import jax
import jax.numpy as jnp
from jax import lax
from jax.experimental import pallas as pl
from jax.experimental.pallas import tpu as pltpu
from jax.experimental.pallas import tpu_sc as plsc

_K = 128
_Y = 16
_C = 128
_N = 320000
_NSEG = _K * _Y
_NC = 2
_NS = 16
_NW = _NC * _NS
_G = 128
_NGROUPS = _N // _G
_BASE_GROUPS = _NGROUPS // _NW
_EXTRA = _NGROUPS - _BASE_GROUPS * _NW
_CG = 1
_CHUNK = _CG * _G
_NCHUNKS = _BASE_GROUPS // _CG
_NBUF = 3
_NLAB = (_BASE_GROUPS + 2) * _G
_SEG_PER_TILE = _NSEG // _NS

assert _NCHUNKS % _NBUF == 0
assert _NGROUPS * _G == _N


def _sc_body(x_hbm, y_hbm, p_hbm, z_hbm, out_hbm,
             segv, yall, rowidx, rows, acc, lsem, sem0, sem1, sem2,
             ssem0, ssem1, ssem2):
    cid = lax.axis_index("c")
    sid = lax.axis_index("s")
    wid = sid * _NC + cid
    sems = (sem0, sem1, sem2)
    ssems = (ssem0, ssem1, ssem2)

    base_w = pl.multiple_of(
        _G * (_BASE_GROUPS * wid + jnp.minimum(wid, _EXTRA)), _G
    )
    nlab = _BASE_GROUPS * _G

    pltpu.async_copy(x_hbm.at[pl.ds(base_w, nlab)],
                     segv.at[pl.ds(0, nlab)], lsem)
    pltpu.async_copy(y_hbm.at[pl.ds(base_w, nlab)],
                     yall.at[pl.ds(0, nlab)], lsem)

    ramp = jnp.arange(16, dtype=jnp.int32)
    for i in range(_NLAB // 16):
        o = i * 16
        rowidx[pl.ds(o, 16)] = ramp + (base_w + o)

    _H = _CHUNK // 2

    def start(c, b):
        pltpu.async_copy(
            p_hbm.at[pl.ds(base_w + c * _CHUNK, _H), :],
            rows.at[b, pl.ds(0, _H), :], sems[b])
        pltpu.async_copy(
            p_hbm.at[rowidx.at[pl.ds(c * _CHUNK + _H, _H)]],
            rows.at[b, pl.ds(_H, _H), :], sems[b])

    def wait_dma(b):
        pltpu.make_async_copy(
            p_hbm.at[pl.ds(0, _H), :],
            rows.at[b, pl.ds(0, _H), :], sems[b]).wait()
        pltpu.make_async_copy(
            p_hbm.at[rowidx.at[pl.ds(0, _H)]],
            rows.at[b, pl.ds(_H, _H), :], sems[b]).wait()

    def fire_scatter(c, b):
        for g in range(_CG):
            pltpu.async_copy(
                rows.at[b, pl.ds(g * _G, _G), :],
                acc.at[segv.at[pl.ds(c * _CHUNK + g * _G, _G)]],
                ssems[b], add=True)

    def drain_scatter(b):
        for g in range(_CG):
            pltpu.make_async_copy(
                rows.at[b, pl.ds(g * _G, _G), :],
                acc.at[segv.at[pl.ds(0, _G)]], ssems[b]).wait()

    pltpu.sync_copy(
        z_hbm.at[pl.ds(sid * _SEG_PER_TILE, _SEG_PER_TILE), :],
        acc.at[pl.ds(sid * _SEG_PER_TILE, _SEG_PER_TILE), :],
    )

    start(0, 0)
    start(1, 1)

    pltpu.make_async_copy(x_hbm.at[pl.ds(0, nlab)],
                          segv.at[pl.ds(0, nlab)], lsem).wait()
    pltpu.make_async_copy(y_hbm.at[pl.ds(0, nlab)],
                          yall.at[pl.ds(0, nlab)], lsem).wait()
    for i in range(nlab // 16):
        o = i * 16
        segv[pl.ds(o, 16)] = segv[pl.ds(o, 16)] * _Y + yall[pl.ds(o, 16)]

    plsc.subcore_barrier()

    def trio_body(j, carry):
        for b in range(_NBUF):
            c = 3 * j + b
            c2 = c + 2
            b2 = (b + 2) % _NBUF

            @pl.when(jnp.logical_and(c2 < _NCHUNKS, c >= 1))
            def _drain_prev():
                drain_scatter(b2)

            @pl.when(c2 < _NCHUNKS)
            def _start_next():
                start(c2, b2)

            wait_dma(b)
            fire_scatter(c, b)
        return carry

    lax.fori_loop(0, _NCHUNKS // _NBUF, trio_body, 0)
    for b in range(_NBUF):
        drain_scatter(b)

    @pl.when(wid < _EXTRA)
    def _tail():
        base = base_w + nlab
        pltpu.sync_copy(x_hbm.at[pl.ds(base, _G)],
                        segv.at[pl.ds(nlab, _G)])
        pltpu.sync_copy(y_hbm.at[pl.ds(base, _G)],
                        yall.at[pl.ds(nlab, _G)])
        pltpu.sync_copy(p_hbm.at[pl.ds(base, _G), :],
                        rows.at[0, pl.ds(0, _G), :])
        for i in range(_G // 16):
            o = nlab + i * 16
            segv[pl.ds(o, 16)] = segv[pl.ds(o, 16)] * _Y + yall[pl.ds(o, 16)]
        pltpu.sync_copy(rows.at[0, pl.ds(0, _G), :],
                        acc.at[segv.at[pl.ds(nlab, _G)]], add=True)

    plsc.subcore_barrier()
    pltpu.sync_copy(
        acc.at[pl.ds(sid * _SEG_PER_TILE, _SEG_PER_TILE), :],
        out_hbm.at[cid, pl.ds(sid * _SEG_PER_TILE, _SEG_PER_TILE), :],
    )


_sc_call = pl.kernel(
    _sc_body,
    out_type=jax.ShapeDtypeStruct((_NC, _NSEG, _C), jnp.float32),
    mesh=plsc.VectorSubcoreMesh(core_axis_name="c", subcore_axis_name="s"),
    scratch_types=[
        pltpu.VMEM((_NLAB,), jnp.int32),
        pltpu.VMEM((_NLAB,), jnp.int32),
        pltpu.VMEM((_NLAB,), jnp.int32),
        pltpu.VMEM((_NBUF, _CHUNK, _C), jnp.float32),
        pltpu.VMEM_SHARED((_NSEG, _C), jnp.float32),
        pltpu.SemaphoreType.DMA,
        pltpu.SemaphoreType.DMA,
        pltpu.SemaphoreType.DMA,
        pltpu.SemaphoreType.DMA,
        pltpu.SemaphoreType.DMA,
        pltpu.SemaphoreType.DMA,
        pltpu.SemaphoreType.DMA,
    ],
)


def _merge_body(em_ref, p_ref, o_ref):
    o_ref[...] = em_ref[...] + p_ref[0] + p_ref[1]


_merge = pl.pallas_call(
    _merge_body,
    out_shape=jax.ShapeDtypeStruct((_NSEG, _C), jnp.float32),
)


def kernel(x_labels, y_labels, posterior_estimate, emission_numerator):
    zeros = jnp.zeros((_NSEG, _C), jnp.float32)
    partials = _sc_call(x_labels, y_labels, posterior_estimate, zeros)
    out = _merge(emission_numerator.reshape(_NSEG, _C), partials)
    return out.reshape(_K, _Y, _C)


# speedup vs baseline: 1.0388x; 1.0388x over previous
import jax
import jax.numpy as jnp
from jax import lax
from jax.experimental import pallas as pl
from jax.experimental.pallas import tpu as pltpu
from jax.experimental.pallas import tpu_sc as plsc

_K = 128
_Y = 16
_C = 128
_N = 320000
_NSEG = _K * _Y
_NC = 2
_NS = 16
_NW = _NC * _NS
_G = 128
_NGROUPS = _N // _G
_BASE_GROUPS = _NGROUPS // _NW
_EXTRA = _NGROUPS - _BASE_GROUPS * _NW
_CG = 1
_CHUNK = _CG * _G
_NCHUNKS = _BASE_GROUPS // _CG
_NBUF = 3
_NLAB = (_BASE_GROUPS + 2) * _G
_SEG_PER_TILE = _NSEG // _NS

assert _NCHUNKS % _NBUF == 0
assert _NGROUPS * _G == _N


def _sc_body(x_hbm, y_hbm, p_hbm, z_hbm, out_hbm,
             segv, yall, rows, acc, lsem, sem0, sem1, sem2,
             ssem0, ssem1, ssem2):
    cid = lax.axis_index("c")
    sid = lax.axis_index("s")
    wid = sid * _NC + cid
    sems = (sem0, sem1, sem2)
    ssems = (ssem0, ssem1, ssem2)

    base_w = pl.multiple_of(
        _G * (_BASE_GROUPS * wid + jnp.minimum(wid, _EXTRA)), _G
    )
    nlab = _BASE_GROUPS * _G

    pltpu.async_copy(x_hbm.at[pl.ds(base_w, nlab)],
                     segv.at[pl.ds(0, nlab)], lsem)
    pltpu.async_copy(y_hbm.at[pl.ds(base_w, nlab)],
                     yall.at[pl.ds(0, nlab)], lsem)

    def start(c, b):
        pltpu.async_copy(p_hbm.at[pl.ds(base_w + c * _CHUNK, _CHUNK), :],
                         rows.at[b], sems[b])

    def wait_dma(b):
        pltpu.make_async_copy(p_hbm.at[pl.ds(0, _CHUNK), :], rows.at[b],
                              sems[b]).wait()

    def fire_scatter(c, b):
        for g in range(_CG):
            pltpu.async_copy(
                rows.at[b, pl.ds(g * _G, _G), :],
                acc.at[segv.at[pl.ds(c * _CHUNK + g * _G, _G)]],
                ssems[b], add=True)

    def drain_scatter(b):
        for g in range(_CG):
            pltpu.make_async_copy(
                rows.at[b, pl.ds(g * _G, _G), :],
                acc.at[segv.at[pl.ds(0, _G)]], ssems[b]).wait()

    pltpu.sync_copy(
        z_hbm.at[pl.ds(sid * _SEG_PER_TILE, _SEG_PER_TILE), :],
        acc.at[pl.ds(sid * _SEG_PER_TILE, _SEG_PER_TILE), :],
    )

    start(0, 0)
    start(1, 1)

    pltpu.make_async_copy(x_hbm.at[pl.ds(0, nlab)],
                          segv.at[pl.ds(0, nlab)], lsem).wait()
    pltpu.make_async_copy(y_hbm.at[pl.ds(0, nlab)],
                          yall.at[pl.ds(0, nlab)], lsem).wait()
    for i in range(nlab // 16):
        o = i * 16
        segv[pl.ds(o, 16)] = segv[pl.ds(o, 16)] * _Y + yall[pl.ds(o, 16)]

    plsc.subcore_barrier()

    def trio_body(j, carry):
        for b in range(_NBUF):
            c = 3 * j + b
            c2 = c + 2
            b2 = (b + 2) % _NBUF

            @pl.when(jnp.logical_and(c2 < _NCHUNKS, c >= 1))
            def _drain_prev():
                drain_scatter(b2)

            @pl.when(c2 < _NCHUNKS)
            def _start_next():
                start(c2, b2)

            wait_dma(b)
            fire_scatter(c, b)
        return carry

    lax.fori_loop(0, _NCHUNKS // _NBUF, trio_body, 0)
    for b in range(_NBUF):
        drain_scatter(b)

    @pl.when(wid < _EXTRA)
    def _tail():
        base = base_w + nlab
        pltpu.sync_copy(x_hbm.at[pl.ds(base, _G)],
                        segv.at[pl.ds(nlab, _G)])
        pltpu.sync_copy(y_hbm.at[pl.ds(base, _G)],
                        yall.at[pl.ds(nlab, _G)])
        pltpu.sync_copy(p_hbm.at[pl.ds(base, _G), :],
                        rows.at[0, pl.ds(0, _G), :])
        for i in range(_G // 16):
            o = nlab + i * 16
            segv[pl.ds(o, 16)] = segv[pl.ds(o, 16)] * _Y + yall[pl.ds(o, 16)]
        pltpu.sync_copy(rows.at[0, pl.ds(0, _G), :],
                        acc.at[segv.at[pl.ds(nlab, _G)]], add=True)

    plsc.subcore_barrier()
    pltpu.sync_copy(
        acc.at[pl.ds(sid * _SEG_PER_TILE, _SEG_PER_TILE), :],
        out_hbm.at[cid, pl.ds(sid * _SEG_PER_TILE, _SEG_PER_TILE), :],
    )


_sc_call = pl.kernel(
    _sc_body,
    out_type=jax.ShapeDtypeStruct((_NC, _NSEG, _C), jnp.float32),
    mesh=plsc.VectorSubcoreMesh(core_axis_name="c", subcore_axis_name="s"),
    scratch_types=[
        pltpu.VMEM((_NLAB,), jnp.int32),
        pltpu.VMEM((_NLAB,), jnp.int32),
        pltpu.VMEM((_NBUF, _CHUNK, _C), jnp.float32),
        pltpu.VMEM_SHARED((_NSEG, _C), jnp.float32),
        pltpu.SemaphoreType.DMA,
        pltpu.SemaphoreType.DMA,
        pltpu.SemaphoreType.DMA,
        pltpu.SemaphoreType.DMA,
        pltpu.SemaphoreType.DMA,
        pltpu.SemaphoreType.DMA,
        pltpu.SemaphoreType.DMA,
    ],
)


def _merge_body(em_ref, p_ref, o_ref):
    o_ref[...] = em_ref[...] + p_ref[0] + p_ref[1]


_merge = pl.pallas_call(
    _merge_body,
    out_shape=jax.ShapeDtypeStruct((_NSEG, _C), jnp.float32),
)


def kernel(x_labels, y_labels, posterior_estimate, emission_numerator):
    zeros = jnp.zeros((_NSEG, _C), jnp.float32)
    partials = _sc_call(x_labels, y_labels, posterior_estimate, zeros)
    out = _merge(emission_numerator.reshape(_NSEG, _C), partials)
    return out.reshape(_K, _Y, _C)
